# baseline (speedup 1.0000x reference)
import math

import jax
import jax.numpy as jnp
from jax import lax
from jax.experimental import pallas as pl
from jax.experimental.pallas import tpu as pltpu
from jax.experimental.pallas import tpu_sc as plsc

NUM_CORES = 2
NUM_SUBCORES = 16
NUM_WORKERS = NUM_CORES * NUM_SUBCORES
LANES = 16
NSLOT = 4


def _emb_body(x_hbm, lut_hbm, out_hbm, idx_v, rows_v, g0, g1, g2, g3, s0,
              s1, s2, s3):
    nb, h = idx_v.shape
    d = rows_v.shape[-1]
    scale = jnp.float32(math.sqrt(d))
    wid = lax.axis_index("s") * NUM_CORES + lax.axis_index("c")
    base = wid * nb
    gsem = (g0, g1, g2, g3)
    ssem = (s0, s1, s2, s3)

    def fire_gather(bi, slot):
        pltpu.async_copy(
            lut_hbm.at[idx_v.at[bi]], rows_v.at[slot], gsem[slot])

    def drain_gather(slot):
        pltpu.make_async_copy(
            lut_hbm.at[idx_v.at[0]], rows_v.at[slot], gsem[slot]).wait()

    def fire_store(bi, slot):
        pltpu.async_copy(
            rows_v.at[slot],
            out_hbm.at[base + bi, pl.ds(0, h), pl.ds(0, d)], ssem[slot])

    def drain_store(slot):
        pltpu.make_async_copy(
            rows_v.at[slot], out_hbm.at[0, pl.ds(0, h), pl.ds(0, d)],
            ssem[slot]).wait()

    def scale_slot(slot):
        def row_body(r, _):
            row = rows_v.at[slot, r]
            for k in range(d // LANES):
                sl = pl.ds(k * LANES, LANES)
                row[sl] = row[sl] * scale
            return 0

        lax.fori_loop(0, h, row_body, 0, unroll=2)

    pltpu.sync_copy(x_hbm.at[pl.ds(base, nb)], idx_v)

    fire_gather(0, 0)
    fire_gather(1, 1)

    def group_body(t, _):
        for p in range(NSLOT):
            bi = t * NSLOT + p
            q = (p + 2) % NSLOT
            if p < 2:
                @pl.when(t >= 1)
                def _():
                    drain_store(q)
                fire_gather(bi + 2, q)
            else:
                drain_store(q)

                @pl.when(t < (nb // NSLOT) - 1)
                def _():
                    fire_gather(bi + 2, q)
            drain_gather(p)
            scale_slot(p)
            fire_store(bi, p)
        return 0

    lax.fori_loop(0, nb // NSLOT, group_body, 0)

    drain_store(2)
    drain_store(3)


def kernel(x, lut):
    b, h = x.shape
    v, d = lut.shape
    nb = b // NUM_WORKERS
    assert b % (NUM_WORKERS * NSLOT) == 0
    assert d % LANES == 0
    hp = (h + 7) // 8 * 8
    dp = 128

    xi = x.astype(jnp.int32)

    mesh = plsc.VectorSubcoreMesh(core_axis_name="c", subcore_axis_name="s")
    run = pl.kernel(
        _emb_body,
        out_type=jax.ShapeDtypeStruct((b, hp, dp), jnp.float32),
        mesh=mesh,
        scratch_types=[
            pltpu.VMEM((nb, h), jnp.int32),
            pltpu.VMEM((NSLOT, h, d), jnp.float32),
        ] + [pltpu.SemaphoreType.DMA] * 8,
        compiler_params=pltpu.CompilerParams(use_tc_tiling_on_sc=False),
    )
    buf = run(xi, lut)
    return buf[:, :h, :d]


# speedup vs baseline: 1.0483x; 1.0483x over previous
import math

import jax
import jax.numpy as jnp
from jax import lax
from jax.experimental import pallas as pl
from jax.experimental.pallas import tpu as pltpu
from jax.experimental.pallas import tpu_sc as plsc

NUM_CORES = 2
NUM_SUBCORES = 16
NUM_WORKERS = NUM_CORES * NUM_SUBCORES
LANES = 16
NSLOT = 4


def _emb_body(x_hbm, lut_hbm, out_hbm, idx_v, rows_v, g0, g1, g2, g3, s0,
              s1, s2, s3):
    nu, h2 = idx_v.shape
    h = h2 // 2
    d = rows_v.shape[-1]
    scale = jnp.float32(math.sqrt(d))
    wid = lax.axis_index("s") * NUM_CORES + lax.axis_index("c")
    base = wid * nu
    gsem = (g0, g1, g2, g3)
    ssem = (s0, s1, s2, s3)

    def fire_gather(u, slot):
        pltpu.async_copy(
            lut_hbm.at[idx_v.at[u]], rows_v.at[slot], gsem[slot])

    def drain_gather(slot):
        pltpu.make_async_copy(
            lut_hbm.at[idx_v.at[0]], rows_v.at[slot], gsem[slot]).wait()

    def fire_store(u, slot):
        for half in range(2):
            pltpu.async_copy(
                rows_v.at[slot, pl.ds(half * h, h)],
                out_hbm.at[(base + u) * 2 + half, pl.ds(0, h), pl.ds(0, d)],
                ssem[slot])

    def drain_store(slot):
        for half in range(2):
            pltpu.make_async_copy(
                rows_v.at[slot, pl.ds(half * h, h)],
                out_hbm.at[0, pl.ds(0, h), pl.ds(0, d)], ssem[slot]).wait()

    def scale_slot(slot):
        def row_body(r, _):
            row = rows_v.at[slot, r]
            for k in range(d // LANES):
                sl = pl.ds(k * LANES, LANES)
                row[sl] = row[sl] * scale
            return 0

        lax.fori_loop(0, h2, row_body, 0, unroll=2)

    pltpu.sync_copy(x_hbm.at[pl.ds(base, nu)], idx_v)

    fire_gather(0, 0)
    fire_gather(1, 1)

    def group_body(t, _):
        for p in range(NSLOT):
            u = t * NSLOT + p
            q = (p + 2) % NSLOT
            if p < 2:
                @pl.when(t >= 1)
                def _():
                    drain_store(q)
                fire_gather(u + 2, q)
            else:
                drain_store(q)

                @pl.when(t < (nu // NSLOT) - 1)
                def _():
                    fire_gather(u + 2, q)
            drain_gather(p)
            scale_slot(p)
            fire_store(u, p)
        return 0

    lax.fori_loop(0, nu // NSLOT, group_body, 0)

    drain_store(2)
    drain_store(3)


def kernel(x, lut):
    b, h = x.shape
    v, d = lut.shape
    nu = b // (2 * NUM_WORKERS)
    assert b % (2 * NUM_WORKERS * NSLOT) == 0
    assert d % LANES == 0
    assert 2 * h <= 128
    hp = (h + 7) // 8 * 8
    dp = 128

    xi = x.astype(jnp.int32).reshape(b // 2, 2 * h)

    mesh = plsc.VectorSubcoreMesh(core_axis_name="c", subcore_axis_name="s")
    run = pl.kernel(
        _emb_body,
        out_type=jax.ShapeDtypeStruct((b, hp, dp), jnp.float32),
        mesh=mesh,
        scratch_types=[
            pltpu.VMEM((nu, 2 * h), jnp.int32),
            pltpu.VMEM((NSLOT, 2 * h, d), jnp.float32),
        ] + [pltpu.SemaphoreType.DMA] * 8,
        compiler_params=pltpu.CompilerParams(use_tc_tiling_on_sc=False),
    )
    buf = run(xi, lut)
    return buf[:, :h, :d]
